# baseline (speedup 1.0000x reference)
import functools

import numpy as np
import jax
import jax.numpy as jnp
from jax import lax
from jax.experimental import pallas as pl
from jax.experimental.pallas import tpu as pltpu
from jax.experimental.pallas import tpu_sc as plsc

N = 10000
D = 128
B = 128

_info = plsc.get_sparse_core_info()
NC = _info.num_cores
NS = _info.num_subcores
NW = NC * NS

CHUNK = (N // (NW * 8)) * 8
REM = N - NW * CHUNK
assert 0 <= REM <= 128 and REM % 8 == 0

_NG = 1
while CHUNK // _NG > 128 or CHUNK % _NG or (CHUNK // _NG) % 8:
    _NG += 1
NGROUPS = _NG
GROUP = CHUNK // NGROUPS

ROWS_PER_SUB = B // NS

_ONES = np.ones((GROUP, D), np.float32)


def _sc_segment_sum(x, batch, ones):
    @functools.partial(
        pl.kernel,
        mesh=plsc.VectorSubcoreMesh(core_axis_name="c", subcore_axis_name="s"),
        out_type=[
            jax.ShapeDtypeStruct((NC * B, D), jnp.float32),
            jax.ShapeDtypeStruct((NC * B, D), jnp.float32),
        ],
        scratch_types=[
            pltpu.VMEM((CHUNK, D), jnp.float32),
            pltpu.VMEM((NGROUPS, GROUP), jnp.int32),
            pltpu.VMEM((GROUP, D), jnp.float32),
            pltpu.VMEM((max(REM, 8), D), jnp.float32),
            pltpu.VMEM((1, max(REM, 8)), jnp.int32),
            pltpu.VMEM((ROWS_PER_SUB, D), jnp.float32),
            pltpu.VMEM_SHARED((B, D), jnp.float32),
            pltpu.VMEM_SHARED((B, D), jnp.float32),
            pltpu.SemaphoreType.DMA,
            pltpu.SemaphoreType.DMA,
            pltpu.SemaphoreType.DMA,
            pltpu.SemaphoreType.DMA,
            pltpu.SemaphoreType.DMA,
        ],
    )
    def k(x_hbm, batch_hbm, ones_hbm, out_hbm, cnt_hbm,
          xbuf, idxbuf, onesbuf, xrem, idxrem, zbuf, acc, cacc,
          sem0, sem1, sem2, semi, semz):
        c = lax.axis_index("c")
        s = lax.axis_index("s")
        wid = s * NC + c
        base = wid * CHUNK

        sems = [sem0, sem1, sem2]
        assert NGROUPS == len(sems)

        xcopies = [
            pltpu.async_copy(
                x_hbm.at[pl.ds(base + g * GROUP, GROUP)],
                xbuf.at[pl.ds(g * GROUP, GROUP)],
                sems[g],
            )
            for g in range(NGROUPS)
        ]
        icopies = [
            pltpu.async_copy(
                batch_hbm.at[pl.ds(base + g * GROUP, GROUP)],
                idxbuf.at[g],
                semi,
            )
            for g in range(NGROUPS)
        ]
        ocopy = pltpu.async_copy(ones_hbm, onesbuf, semi)

        zvec = jnp.zeros((16,), jnp.float32)
        for r in range(ROWS_PER_SUB):
            for ch in range(D // 16):
                zbuf[r, pl.ds(ch * 16, 16)] = zvec
        row = s * ROWS_PER_SUB
        zc0 = pltpu.async_copy(zbuf, acc.at[pl.ds(row, ROWS_PER_SUB)], semz)
        zc1 = pltpu.async_copy(zbuf, cacc.at[pl.ds(row, ROWS_PER_SUB)], semz)

        for cp in icopies:
            cp.wait()
        ocopy.wait()
        zc0.wait()
        zc1.wait()
        plsc.subcore_barrier()

        scopies = []
        for g in range(NGROUPS):
            xcopies[g].wait()
            scopies.append(
                pltpu.async_copy(
                    xbuf.at[pl.ds(g * GROUP, GROUP)],
                    acc.at[idxbuf.at[g]],
                    semz,
                    add=True,
                )
            )
            scopies.append(
                pltpu.async_copy(
                    onesbuf,
                    cacc.at[idxbuf.at[g]],
                    semz,
                    add=True,
                )
            )
        for cp in scopies:
            cp.wait()

        if REM:
            @pl.when(wid == 0)
            def _():
                pltpu.sync_copy(
                    x_hbm.at[pl.ds(NW * CHUNK, REM)], xrem.at[pl.ds(0, REM)]
                )
                pltpu.sync_copy(
                    batch_hbm.at[pl.ds(NW * CHUNK, REM)],
                    idxrem.at[0, pl.ds(0, REM)],
                )
                pltpu.sync_copy(
                    xrem.at[pl.ds(0, REM)],
                    acc.at[idxrem.at[0, pl.ds(0, REM)]],
                    add=True,
                )
                pltpu.sync_copy(
                    onesbuf.at[pl.ds(0, REM)],
                    cacc.at[idxrem.at[0, pl.ds(0, REM)]],
                    add=True,
                )

        plsc.subcore_barrier()

        pltpu.sync_copy(
            acc.at[pl.ds(row, ROWS_PER_SUB)],
            out_hbm.at[pl.ds(c * B + row, ROWS_PER_SUB)],
        )
        pltpu.sync_copy(
            cacc.at[pl.ds(row, ROWS_PER_SUB)],
            cnt_hbm.at[pl.ds(c * B + row, ROWS_PER_SUB)],
        )

    return k(x, batch, ones)


def _tc_finish(partials, cnt, u, w, bias):
    def body(p_ref, c_ref, u_ref, w_ref, b_ref, o_ref):
        sums = p_ref[pl.ds(0, B), :] + p_ref[pl.ds(B, B), :]
        counts = c_ref[pl.ds(0, B), pl.ds(0, 1)] + c_ref[pl.ds(B, B), pl.ds(0, 1)]
        mean = sums / jnp.maximum(counts, 1.0)
        out = lax.dot_general(
            u_ref[...], w_ref[:, pl.ds(0, D)],
            (((1,), (1,)), ((), ())), preferred_element_type=jnp.float32,
        )
        out = out + lax.dot_general(
            mean, w_ref[:, pl.ds(D, D)],
            (((1,), (1,)), ((), ())), preferred_element_type=jnp.float32,
        )
        out = out + b_ref[...]
        o_ref[...] = jnp.maximum(out, 0.0)

    return pl.pallas_call(
        body,
        out_shape=jax.ShapeDtypeStruct((B, D), jnp.float32),
    )(partials, cnt, u, w, bias)


@jax.jit
def kernel(x, edge_index, edge_attr, u, batch, W, b):
    del edge_index, edge_attr
    batch = batch.astype(jnp.int32)
    partials, cnt = _sc_segment_sum(x, batch, jnp.asarray(_ONES))
    bias = b.reshape(1, D)
    return _tc_finish(partials, cnt, u, W, bias)
